# Initial kernel scaffold; baseline (speedup 1.0000x reference)
import jax
import jax.numpy as jnp
from jax.experimental import pallas as pl


def kernel(x, edge_index, W1, b1, g1, be1, W2, b2, g2, be2, W3, b3):
    raise NotImplementedError("write your pallas kernel here")


# speedup vs baseline: 10.6987x; 10.6987x over previous
import functools

import jax
import jax.numpy as jnp
from jax import lax
from jax.experimental import pallas as pl
from jax.experimental.pallas import tpu as pltpu
from jax.experimental.pallas import tpu_sc as plsc

ALPHA = 0.5
NC = 2
NS = 16
C = 80


def _pad_rows(n):
    per = -(-n // NS)
    per = -(-per // 8) * 8
    return per * NS, per


@functools.lru_cache(maxsize=None)
def _make_deg_kernel(n, e):
    ep = e // (NC * NS)
    nchunks = ep // C
    np_, rows = _pad_rows(n)
    mesh = plsc.VectorSubcoreMesh(core_axis_name="c", subcore_axis_name="s")

    @functools.partial(
        pl.kernel,
        out_type=jax.ShapeDtypeStruct((NC, np_, 1), jnp.float32),
        mesh=mesh,
        scratch_types=[
            pltpu.VMEM_SHARED((np_, 1), jnp.float32),
            pltpu.VMEM((C,), jnp.int32),
            pltpu.VMEM((C, 1), jnp.float32),
        ],
    )
    def deg_kernel(dst_ix, zeros1, ones, out, deg_sh, dstbuf, onesbuf):
        c = lax.axis_index("c")
        s = lax.axis_index("s")
        r0 = pl.multiple_of(s * rows, 8)
        pltpu.sync_copy(zeros1.at[pl.ds(r0, rows), :], deg_sh.at[pl.ds(r0, rows), :])
        pltpu.sync_copy(ones, onesbuf)
        plsc.subcore_barrier()
        base = c * (e // NC) + s * ep

        def chunk(j, carry):
            off = pl.multiple_of(base + j * C, 8)
            pltpu.sync_copy(dst_ix.at[pl.ds(off, C)], dstbuf)
            pltpu.sync_copy(onesbuf, deg_sh.at[dstbuf], add=True)
            return carry

        lax.fori_loop(0, nchunks, chunk, 0)
        plsc.subcore_barrier()
        pltpu.sync_copy(deg_sh.at[pl.ds(r0, rows), :], out.at[c, pl.ds(r0, rows), :])

    return deg_kernel


@functools.lru_cache(maxsize=None)
def _make_agg_kernel(n, d, e):
    ep = e // (NC * NS)
    nchunks = ep // C
    np_, rows = _pad_rows(n)
    mesh = plsc.VectorSubcoreMesh(core_axis_name="c", subcore_axis_name="s")

    @functools.partial(
        pl.kernel,
        out_type=jax.ShapeDtypeStruct((NC, np_, d), jnp.float32),
        mesh=mesh,
        scratch_types=[
            pltpu.VMEM_SHARED((np_, d), jnp.float32),
            pltpu.VMEM((C,), jnp.int32),
            pltpu.VMEM((C,), jnp.int32),
            pltpu.VMEM((C, d), jnp.float32),
            pltpu.SemaphoreType.DMA,
        ],
    )
    def agg_kernel(h, src_ix, dst_ix, zeros, out, agg_sh, srcbuf, dstbuf, rowbuf, sem):
        c = lax.axis_index("c")
        s = lax.axis_index("s")
        r0 = pl.multiple_of(s * rows, 8)
        pltpu.sync_copy(zeros.at[pl.ds(r0, rows), :], agg_sh.at[pl.ds(r0, rows), :])
        plsc.subcore_barrier()
        base = c * (e // NC) + s * ep

        def chunk(j, carry):
            off = pl.multiple_of(base + j * C, 8)
            pltpu.sync_copy(src_ix.at[pl.ds(off, C)], srcbuf)
            pltpu.sync_copy(dst_ix.at[pl.ds(off, C)], dstbuf)
            pltpu.async_copy(h.at[srcbuf], rowbuf, sem).wait()
            pltpu.sync_copy(rowbuf, agg_sh.at[dstbuf], add=True)
            return carry

        lax.fori_loop(0, nchunks, chunk, 0)
        plsc.subcore_barrier()
        pltpu.sync_copy(agg_sh.at[pl.ds(r0, rows), :], out.at[c, pl.ds(r0, rows), :])

    return agg_kernel


_R = 1000


def _tc1_body(deg2, x, w, dinv_o, h_o):
    deg = deg2[0] + deg2[1] + 1.0
    dinv = lax.rsqrt(deg)
    dinv_o[...] = dinv
    h = lax.dot_general(x[...], w[...], (((1,), (1,)), ((), ())),
                        preferred_element_type=jnp.float32)
    h_o[...] = h * dinv


def _tc_mid_body(aggp, hprev, xres, dinv, b, g, be, w, x_o, h_o):
    agg = aggp[0] + aggp[1]
    dv = dinv[...]
    h = dv * (agg + hprev[...]) + b[...][None, :]
    mu = jnp.mean(h, axis=-1, keepdims=True)
    var = jnp.mean((h - mu) ** 2, axis=-1, keepdims=True)
    ln = (h - mu) * lax.rsqrt(var + 1e-5) * g[...][None, :] + be[...][None, :]
    xn = jnp.maximum(ALPHA * ln + (1.0 - ALPHA) * xres[...], 0.0)
    x_o[...] = xn
    h2 = lax.dot_general(xn, w[...], (((1,), (1,)), ((), ())),
                         preferred_element_type=jnp.float32)
    h_o[...] = h2 * dv


def _tc_out_body(aggp, hprev, dinv, b, out_o):
    agg = aggp[0] + aggp[1]
    out_o[...] = dinv[...] * (agg + hprev[...]) + b[...][None, :]


@functools.lru_cache(maxsize=None)
def _make_tc_kernels(n, d):
    grid = (n // _R,)
    bpart = pl.BlockSpec((NC, _R, d), lambda i: (0, i, 0))
    bpcol = pl.BlockSpec((NC, _R, 1), lambda i: (0, i, 0))
    brow = pl.BlockSpec((_R, d), lambda i: (i, 0))
    bcol = pl.BlockSpec((_R, 1), lambda i: (i, 0))
    bvec = pl.BlockSpec((d,), lambda i: (0,))
    bmat = pl.BlockSpec((d, d), lambda i: (0, 0))
    f32 = jnp.float32

    tc1 = pl.pallas_call(
        _tc1_body,
        grid=grid,
        in_specs=[bpcol, brow, bmat],
        out_specs=[bcol, brow],
        out_shape=[jax.ShapeDtypeStruct((n, 1), f32),
                   jax.ShapeDtypeStruct((n, d), f32)],
    )
    tc_mid = pl.pallas_call(
        _tc_mid_body,
        grid=grid,
        in_specs=[bpart, brow, brow, bcol, bvec, bvec, bvec, bmat],
        out_specs=[brow, brow],
        out_shape=[jax.ShapeDtypeStruct((n, d), f32),
                   jax.ShapeDtypeStruct((n, d), f32)],
    )
    tc_out = pl.pallas_call(
        _tc_out_body,
        grid=grid,
        in_specs=[bpart, brow, bcol, bvec],
        out_specs=brow,
        out_shape=jax.ShapeDtypeStruct((n, d), f32),
    )
    return tc1, tc_mid, tc_out


def kernel(x, edge_index, W1, b1, g1, be1, W2, b2, g2, be2, W3, b3):
    n, d = x.shape
    e = edge_index.shape[1]
    np_, _ = _pad_rows(n)
    deg_k = _make_deg_kernel(n, e)
    agg_k = _make_agg_kernel(n, d, e)
    tc1, tc_mid, tc_out = _make_tc_kernels(n, d)

    src_ix = edge_index[0]
    dst_ix = edge_index[1]
    zeros = jnp.zeros((np_, d), jnp.float32)
    zeros1 = jnp.zeros((np_, 1), jnp.float32)
    ones = jnp.ones((C, 1), jnp.float32)

    deg2 = deg_k(dst_ix, zeros1, ones)
    dinv, h1 = tc1(deg2, x, W1)
    p1 = agg_k(h1, src_ix, dst_ix, zeros)
    x1, h2 = tc_mid(p1, h1, x, dinv, b1, g1, be1, W2)
    p2 = agg_k(h2, src_ix, dst_ix, zeros)
    x2, h3 = tc_mid(p2, h2, x1, dinv, b2, g2, be2, W3)
    p3 = agg_k(h3, src_ix, dst_ix, zeros)
    return tc_out(p3, h3, dinv, b3)


# speedup vs baseline: 19.1154x; 1.7867x over previous
import functools

import jax
import jax.numpy as jnp
from jax import lax
from jax.experimental import pallas as pl
from jax.experimental.pallas import tpu as pltpu
from jax.experimental.pallas import tpu_sc as plsc

ALPHA = 0.5
NC = 2
NS = 16
C = 80


def _pad_rows(n):
    per = -(-n // NS)
    per = -(-per // 8) * 8
    return per * NS, per


_NB = 4


@functools.lru_cache(maxsize=None)
def _make_deg_kernel(n, e):
    ep = e // (NC * NS)
    nchunks = ep // C
    np_, rows = _pad_rows(n)
    mesh = plsc.VectorSubcoreMesh(core_axis_name="c", subcore_axis_name="s")

    @functools.partial(
        pl.kernel,
        out_type=jax.ShapeDtypeStruct((NC, np_, 1), jnp.float32),
        mesh=mesh,
        scratch_types=[
            pltpu.VMEM_SHARED((np_, 1), jnp.float32),
            pltpu.VMEM((C, 1), jnp.float32),
        ] + [pltpu.VMEM((C,), jnp.int32) for _ in range(_NB)]
          + [pltpu.SemaphoreType.DMA for _ in range(_NB)],
    )
    def deg_kernel(dst_ix, zeros1, ones, out, deg_sh, onesbuf, *dbsx):
        dbs, sixs = dbsx[:_NB], dbsx[_NB:]
        c = lax.axis_index("c")
        s = lax.axis_index("s")
        r0 = pl.multiple_of(s * rows, 8)
        pltpu.sync_copy(zeros1.at[pl.ds(r0, rows), :], deg_sh.at[pl.ds(r0, rows), :])
        pltpu.sync_copy(ones, onesbuf)
        plsc.subcore_barrier()
        base = c * (e // NC) + s * ep

        six, ssc = sixs[0], sixs[1]

        def group(g, carry):
            j0 = g * _NB
            for b in range(_NB):
                off = pl.multiple_of(base + (j0 + b) * C, 8)
                pltpu.async_copy(dst_ix.at[pl.ds(off, C)], dbs[b], six)
            for b in range(_NB):
                off = pl.multiple_of(base + (j0 + b) * C, 8)
                pltpu.make_async_copy(dst_ix.at[pl.ds(off, C)], dbs[b], six).wait()
            for b in range(_NB):
                pltpu.async_copy(onesbuf, deg_sh.at[dbs[b]], ssc, add=True)
            for b in range(_NB):
                pltpu.make_async_copy(onesbuf, deg_sh.at[dbs[b]], ssc).wait()
            return carry

        lax.fori_loop(0, nchunks // _NB, group, 0)
        for j in range(nchunks - nchunks % _NB, nchunks):
            off = pl.multiple_of(base + j * C, 8)
            pltpu.sync_copy(dst_ix.at[pl.ds(off, C)], dbs[0])
            pltpu.sync_copy(onesbuf, deg_sh.at[dbs[0]], add=True)
        plsc.subcore_barrier()
        pltpu.sync_copy(deg_sh.at[pl.ds(r0, rows), :], out.at[c, pl.ds(r0, rows), :])

    return deg_kernel


@functools.lru_cache(maxsize=None)
def _make_agg_kernel(n, d, e):
    ep = e // (NC * NS)
    nchunks = ep // C
    np_, rows = _pad_rows(n)
    mesh = plsc.VectorSubcoreMesh(core_axis_name="c", subcore_axis_name="s")


    @functools.partial(
        pl.kernel,
        out_type=jax.ShapeDtypeStruct((NC, np_, d), jnp.float32),
        mesh=mesh,
        scratch_types=[
            pltpu.VMEM_SHARED((np_, d), jnp.float32),
        ] + [pltpu.VMEM((C,), jnp.int32) for _ in range(2 * _NB)]
          + [pltpu.VMEM((C, d), jnp.float32) for _ in range(_NB)]
          + [pltpu.SemaphoreType.DMA for _ in range(2 * _NB)],
    )
    def agg_kernel(h, src_ix, dst_ix, zeros, out, agg_sh, *rest):
        sbs = rest[:_NB]
        dbs = rest[_NB:2 * _NB]
        rbs = rest[2 * _NB:3 * _NB]
        sixs = rest[3 * _NB:4 * _NB]
        sgs = rest[4 * _NB:]
        c = lax.axis_index("c")
        s = lax.axis_index("s")
        r0 = pl.multiple_of(s * rows, 8)
        pltpu.sync_copy(zeros.at[pl.ds(r0, rows), :], agg_sh.at[pl.ds(r0, rows), :])
        base = c * (e // NC) + s * ep
        plsc.subcore_barrier()

        six, sg, ssc = sgs[0], sgs[1], sgs[2]

        def group(g, carry):
            j0 = g * _NB
            for b in range(_NB):
                off = pl.multiple_of(base + (j0 + b) * C, 8)
                pltpu.async_copy(src_ix.at[pl.ds(off, C)], sbs[b], six)
                pltpu.async_copy(dst_ix.at[pl.ds(off, C)], dbs[b], six)
            for b in range(_NB):
                off = pl.multiple_of(base + (j0 + b) * C, 8)
                pltpu.make_async_copy(src_ix.at[pl.ds(off, C)], sbs[b], six).wait()
                pltpu.make_async_copy(dst_ix.at[pl.ds(off, C)], dbs[b], six).wait()
            for b in range(_NB):
                pltpu.async_copy(h.at[sbs[b]], rbs[b], sg)
            for b in range(_NB):
                pltpu.make_async_copy(h.at[sbs[b]], rbs[b], sg).wait()
            for b in range(_NB):
                pltpu.async_copy(rbs[b], agg_sh.at[dbs[b]], ssc, add=True)
            for b in range(_NB):
                pltpu.make_async_copy(rbs[b], agg_sh.at[dbs[b]], ssc).wait()
            return carry

        lax.fori_loop(0, nchunks // _NB, group, 0)
        for j in range(nchunks - nchunks % _NB, nchunks):
            off = pl.multiple_of(base + j * C, 8)
            pltpu.sync_copy(src_ix.at[pl.ds(off, C)], sbs[0])
            pltpu.sync_copy(dst_ix.at[pl.ds(off, C)], dbs[0])
            pltpu.async_copy(h.at[sbs[0]], rbs[0], sgs[3]).wait()
            pltpu.sync_copy(rbs[0], agg_sh.at[dbs[0]], add=True)
        plsc.subcore_barrier()
        pltpu.sync_copy(agg_sh.at[pl.ds(r0, rows), :], out.at[c, pl.ds(r0, rows), :])

    return agg_kernel


_R = 1000


def _tc1_body(deg2, x, w, dinv_o, h_o):
    deg = deg2[0] + deg2[1] + 1.0
    dinv = lax.rsqrt(deg)
    dinv_o[...] = dinv
    h = lax.dot_general(x[...], w[...], (((1,), (1,)), ((), ())),
                        preferred_element_type=jnp.float32)
    h_o[...] = h * dinv


def _tc_mid_body(aggp, hprev, xres, dinv, b, g, be, w, x_o, h_o):
    agg = aggp[0] + aggp[1]
    dv = dinv[...]
    h = dv * (agg + hprev[...]) + b[...][None, :]
    mu = jnp.mean(h, axis=-1, keepdims=True)
    var = jnp.mean((h - mu) ** 2, axis=-1, keepdims=True)
    ln = (h - mu) * lax.rsqrt(var + 1e-5) * g[...][None, :] + be[...][None, :]
    xn = jnp.maximum(ALPHA * ln + (1.0 - ALPHA) * xres[...], 0.0)
    x_o[...] = xn
    h2 = lax.dot_general(xn, w[...], (((1,), (1,)), ((), ())),
                         preferred_element_type=jnp.float32)
    h_o[...] = h2 * dv


def _tc_out_body(aggp, hprev, dinv, b, out_o):
    agg = aggp[0] + aggp[1]
    out_o[...] = dinv[...] * (agg + hprev[...]) + b[...][None, :]


@functools.lru_cache(maxsize=None)
def _make_tc_kernels(n, d):
    grid = (n // _R,)
    bpart = pl.BlockSpec((NC, _R, d), lambda i: (0, i, 0))
    bpcol = pl.BlockSpec((NC, _R, 1), lambda i: (0, i, 0))
    brow = pl.BlockSpec((_R, d), lambda i: (i, 0))
    bcol = pl.BlockSpec((_R, 1), lambda i: (i, 0))
    bvec = pl.BlockSpec((d,), lambda i: (0,))
    bmat = pl.BlockSpec((d, d), lambda i: (0, 0))
    f32 = jnp.float32

    tc1 = pl.pallas_call(
        _tc1_body,
        grid=grid,
        in_specs=[bpcol, brow, bmat],
        out_specs=[bcol, brow],
        out_shape=[jax.ShapeDtypeStruct((n, 1), f32),
                   jax.ShapeDtypeStruct((n, d), f32)],
    )
    tc_mid = pl.pallas_call(
        _tc_mid_body,
        grid=grid,
        in_specs=[bpart, brow, brow, bcol, bvec, bvec, bvec, bmat],
        out_specs=[brow, brow],
        out_shape=[jax.ShapeDtypeStruct((n, d), f32),
                   jax.ShapeDtypeStruct((n, d), f32)],
    )
    tc_out = pl.pallas_call(
        _tc_out_body,
        grid=grid,
        in_specs=[bpart, brow, bcol, bvec],
        out_specs=brow,
        out_shape=jax.ShapeDtypeStruct((n, d), f32),
    )
    return tc1, tc_mid, tc_out


def kernel(x, edge_index, W1, b1, g1, be1, W2, b2, g2, be2, W3, b3):
    n, d = x.shape
    e = edge_index.shape[1]
    np_, _ = _pad_rows(n)
    deg_k = _make_deg_kernel(n, e)
    agg_k = _make_agg_kernel(n, d, e)
    tc1, tc_mid, tc_out = _make_tc_kernels(n, d)

    src_ix = edge_index[0]
    dst_ix = edge_index[1]
    zeros = jnp.zeros((np_, d), jnp.float32)
    zeros1 = jnp.zeros((np_, 1), jnp.float32)
    ones = jnp.ones((C, 1), jnp.float32)

    deg2 = deg_k(dst_ix, zeros1, ones)
    dinv, h1 = tc1(deg2, x, W1)
    p1 = agg_k(h1, src_ix, dst_ix, zeros)
    x1, h2 = tc_mid(p1, h1, x, dinv, b1, g1, be1, W2)
    p2 = agg_k(h2, src_ix, dst_ix, zeros)
    x2, h3 = tc_mid(p2, h2, x1, dinv, b2, g2, be2, W3)
    p3 = agg_k(h3, src_ix, dst_ix, zeros)
    return tc_out(p3, h3, dinv, b3)
